# baseline (speedup 1.0000x reference)
import functools
import math

import numpy as np
import jax
import jax.numpy as jnp
from jax import lax
from jax.experimental import pallas as pl
from jax.experimental.pallas import tpu as pltpu
from jax.experimental.pallas import tpu_sc as plsc

_N_NODES = 10000
_N_EDGES = 320000
_E_KEEP = _N_EDGES // 2
_N_PAD = 10240
_CHUNK = 128
_N_CHUNKS = _E_KEEP // _CHUNK
_NW = 32
_BR = 1024
_WIN = 1280
_NWIN = _N_PAD // _WIN
_RPT = _WIN // 16

_KEEP = [
    np.asarray(np.random.default_rng(s).permutation(_N_EDGES)[:_E_KEEP],
               dtype=np.int32)
    for s in range(3)
]


def _prep_weights(Wq, bq, Wk, bk, Wv, bv, c3_pad):
  c_in = Wq.shape[1] // 2

  def split(W):
    Wi = W[:, :c_in]
    Wj = W[:, c_in:]
    return (Wi - Wj).T, Wj.T

  Aq, Bq = split(Wq)
  Ak, Bk = split(Wk)
  Av, Bv = split(Wv)
  pad = c3_pad - 3 * Wq.shape[0]
  zpad = jnp.zeros((c_in, pad), jnp.float32)
  Wcat = jnp.concatenate([Aq, Ak, Av, zpad, Bq, Bk, Bv, zpad], axis=1)
  bcat = jnp.concatenate(
      [bq, bk, bv, jnp.zeros((pad,), jnp.float32)]).reshape(1, -1)
  return Wcat, bcat


def _proj_body(x_ref, w_ref, b_ref, u_ref, v_ref):
  y = jnp.dot(x_ref[...], w_ref[...], preferred_element_type=jnp.float32)
  half = u_ref.shape[1]
  u_ref[...] = y[:, :half] + b_ref[...]
  v_ref[...] = y[:, half:]


def _proj(xpad, Wcat, bcat):
  c_in = xpad.shape[1]
  c3 = Wcat.shape[1] // 2
  grid = _N_PAD // _BR
  return pl.pallas_call(
      _proj_body,
      grid=(grid,),
      in_specs=[
          pl.BlockSpec((_BR, c_in), lambda i: (i, 0)),
          pl.BlockSpec((c_in, 2 * c3), lambda i: (0, 0)),
          pl.BlockSpec((1, c3), lambda i: (0, 0)),
      ],
      out_specs=[
          pl.BlockSpec((_BR, c3), lambda i: (i, 0)),
          pl.BlockSpec((_BR, c3), lambda i: (i, 0)),
      ],
      out_shape=[
          jax.ShapeDtypeStruct((_N_PAD, c3), jnp.float32),
          jax.ShapeDtypeStruct((_N_PAD, c3), jnp.float32),
      ],
  )(xpad, Wcat, bcat)


def _fin_proj_body(p_ref, c_ref, w_ref, b_ref, x_ref, u_ref, v_ref, *, cp):
  s = p_ref[0] + p_ref[1]
  cnt = (c_ref[0, :, 0] + c_ref[1, :, 0])[:, None]
  h = jnp.where(cnt > 0, s[:, :cp] / jnp.maximum(cnt, 1.0), 0.0)
  h = jnp.maximum(h, 0.0)
  x_ref[...] = h
  y = jnp.dot(h, w_ref[...], preferred_element_type=jnp.float32)
  half = u_ref.shape[1]
  u_ref[...] = y[:, :half] + b_ref[...]
  v_ref[...] = y[:, half:]


def _fin_proj(part, cnt, Wcat, bcat, cp):
  c3 = Wcat.shape[1] // 2
  grid = _N_PAD // _BR
  return pl.pallas_call(
      functools.partial(_fin_proj_body, cp=cp),
      grid=(grid,),
      in_specs=[
          pl.BlockSpec((2, _BR, 128), lambda i: (0, i, 0)),
          pl.BlockSpec((2, _BR, 128), lambda i: (0, i, 0)),
          pl.BlockSpec((cp, 2 * c3), lambda i: (0, 0)),
          pl.BlockSpec((1, c3), lambda i: (0, 0)),
      ],
      out_specs=[
          pl.BlockSpec((_BR, cp), lambda i: (i, 0)),
          pl.BlockSpec((_BR, c3), lambda i: (i, 0)),
          pl.BlockSpec((_BR, c3), lambda i: (i, 0)),
      ],
      out_shape=[
          jax.ShapeDtypeStruct((_N_PAD, cp), jnp.float32),
          jax.ShapeDtypeStruct((_N_PAD, c3), jnp.float32),
          jax.ShapeDtypeStruct((_N_PAD, c3), jnp.float32),
      ],
  )(part, cnt, Wcat, bcat)


def _fin_res_body(p_ref, c_ref, x0_ref, o_ref):
  s = p_ref[0] + p_ref[1]
  cnt = (c_ref[0, :, 0] + c_ref[1, :, 0])[:, None]
  h = jnp.where(cnt > 0, s / jnp.maximum(cnt, 1.0), 0.0)
  o_ref[...] = jnp.maximum(h + x0_ref[...], 0.0)


def _fin_res(part, cnt, x0):
  grid = _N_PAD // _BR
  return pl.pallas_call(
      _fin_res_body,
      grid=(grid,),
      in_specs=[
          pl.BlockSpec((2, _BR, 128), lambda i: (0, i, 0)),
          pl.BlockSpec((2, _BR, 128), lambda i: (0, i, 0)),
          pl.BlockSpec((_BR, 128), lambda i: (i, 0)),
      ],
      out_specs=pl.BlockSpec((_BR, 128), lambda i: (i, 0)),
      out_shape=jax.ShapeDtypeStruct((_N_PAD, 128), jnp.float32),
  )(part, cnt, x0)


def _make_edge_kernel(c_out, hs, c3_pad):
  n_vreg = c_out // 16
  inv = 1.0 / math.sqrt(hs)
  mesh = plsc.VectorSubcoreMesh(core_axis_name="c", subcore_axis_name="s")

  @functools.partial(
      pl.kernel,
      mesh=mesh,
      out_type=[
          jax.ShapeDtypeStruct((2, _N_PAD, 128), jnp.float32),
          jax.ShapeDtypeStruct((2, _N_PAD, 128), jnp.float32),
          jax.ShapeDtypeStruct((_E_KEEP, 128), jnp.float32),
      ],
      scratch_types=[
          pltpu.VMEM((_CHUNK,), jnp.int32),
          pltpu.VMEM((_CHUNK,), jnp.int32),
          pltpu.VMEM((_CHUNK,), jnp.int32),
          pltpu.VMEM((_CHUNK,), jnp.int32),
          pltpu.VMEM((_CHUNK, c3_pad), jnp.float32),
          pltpu.VMEM((_CHUNK, c3_pad), jnp.float32),
          pltpu.VMEM((_CHUNK, 128), jnp.float32),
          pltpu.VMEM((_CHUNK + 16,), jnp.float32),
          pltpu.VMEM_SHARED((_WIN + 8, 128), jnp.float32),
          pltpu.SemaphoreType.DMA,
          pltpu.SemaphoreType.DMA,
      ],
  )
  def edge_kernel(u_hbm, v_hbm, src_hbm, dst_hbm, keep_hbm,
                  part_hbm, cnt_hbm, ctx_hbm,
                  kpos_v, src_v, dst_v, dst_i, u_b, v_b, ctx_b, w_b, acc,
                  sem1, sem2):
    cid = lax.axis_index("c")
    sid = lax.axis_index("s")
    wid = cid * 16 + sid
    zero16 = jnp.zeros((16,), jnp.float32)
    onesv = jnp.where(lax.iota(jnp.int32, 16) == 0,
                      jnp.float32(1.0), jnp.float32(0.0))

    def zrow(r, carry):
      for c in range(8):
        ctx_b[r, pl.ds(c * 16, 16)] = zero16
      return carry

    def zero_acc():
      pltpu.sync_copy(ctx_b, acc.at[pl.ds(sid * _RPT, _CHUNK)])
      @pl.when(sid == 15)
      def _():
        pltpu.sync_copy(ctx_b.at[pl.ds(0, 8)], acc.at[pl.ds(_WIN, 8)])

    n_j = jnp.where(wid < _N_CHUNKS % _NW,
                    _N_CHUNKS // _NW + 1, _N_CHUNKS // _NW)

    lane = lax.iota(jnp.int32, 16)
    perm_idx = []
    st = hs // 2
    while st >= 1:
      perm_idx.append(jnp.bitwise_xor(lane, st))
      st //= 2

    def local_idx(wbase):
      for g in range(_CHUNK // 16):
        dv = dst_v[pl.ds(g * 16, 16)]
        loc = dv - wbase
        ok = jnp.logical_and(loc >= 0, loc < _WIN)
        dst_i[pl.ds(g * 16, 16)] = jnp.where(ok, loc, _WIN)

    def edge_body(e, ecarry):
      w = w_b[pl.ds(e, 16)][0]
      for h in range(n_vreg):
        q = u_b[e, pl.ds(h * 16, 16)] + v_b[e, pl.ds(h * 16, 16)]
        k = (u_b[e, pl.ds(c_out + h * 16, 16)]
             + v_b[e, pl.ds(c_out + h * 16, 16)])
        v = (u_b[e, pl.ds(2 * c_out + h * 16, 16)]
             + v_b[e, pl.ds(2 * c_out + h * 16, 16)])
        s = q * k * inv
        m = s
        for ix in perm_idx:
          m = jnp.maximum(m, m.at[ix].get(mode="promise_in_bounds"))
        p = jnp.exp(s - m)
        z = p
        for ix in perm_idx:
          z = z + z.at[ix].get(mode="promise_in_bounds")
        ctx_b[e, pl.ds(h * 16, 16)] = p * v * (w / z)
      return ecarry

    def win_body(w, carry):
      wbase = pl.multiple_of(w * _WIN, _WIN)
      lax.fori_loop(0, _CHUNK, zrow, 0)
      zero_acc()
      plsc.subcore_barrier()

      def chunk_body(j, carry):
        base = (j * _NW + wid) * _CHUNK
        pltpu.sync_copy(keep_hbm.at[pl.ds(base, _CHUNK)], kpos_v)
        pltpu.async_copy(dst_hbm.at[kpos_v], dst_v, sem2).wait()

        @pl.when(w == 0)
        def _():
          pltpu.async_copy(src_hbm.at[kpos_v], src_v, sem1).wait()
          h1 = pltpu.async_copy(u_hbm.at[dst_v], u_b, sem1)
          h2 = pltpu.async_copy(v_hbm.at[src_v], v_b, sem2)
          h1.wait()
          h2.wait()
          for g in range(_CHUNK // 16):
            sv = src_v[pl.ds(g * 16, 16)]
            dv = dst_v[pl.ds(g * 16, 16)]
            dvec = jnp.abs(dv - sv)
            w_b[pl.ds(g * 16, 16)] = jnp.where(
                dvec > 8, jnp.float32(1.0),
                jnp.where(dvec < 8, jnp.float32(-1.0), jnp.float32(0.0)))
          lax.fori_loop(0, _CHUNK, edge_body, 0)
          pltpu.sync_copy(ctx_b, ctx_hbm.at[pl.ds(base, _CHUNK)])

        @pl.when(w != 0)
        def _():
          pltpu.sync_copy(ctx_hbm.at[pl.ds(base, _CHUNK)], ctx_b)

        local_idx(wbase)
        pltpu.sync_copy(ctx_b, acc.at[dst_i], add=True)
        return carry

      lax.fori_loop(0, n_j, chunk_body, 0)
      plsc.subcore_barrier()
      pltpu.sync_copy(acc.at[pl.ds(sid * _RPT, _RPT)],
                      part_hbm.at[cid].at[pl.ds(wbase + sid * _RPT, _RPT)])
      plsc.subcore_barrier()
      return carry

    lax.fori_loop(0, _NWIN, win_body, 0)

    def cnt_win_body(w, carry):
      wbase = pl.multiple_of(w * _WIN, _WIN)
      lax.fori_loop(0, _CHUNK, zrow, 0)
      zero_acc()

      def onesrow(r, carry):
        ctx_b[r, pl.ds(0, 16)] = onesv
        return carry

      lax.fori_loop(0, _CHUNK, onesrow, 0)
      plsc.subcore_barrier()

      def cnt_body(j, carry):
        base = (j * _NW + wid) * _CHUNK
        pltpu.sync_copy(keep_hbm.at[pl.ds(base, _CHUNK)], kpos_v)
        pltpu.async_copy(dst_hbm.at[kpos_v], dst_v, sem2).wait()
        local_idx(wbase)
        pltpu.sync_copy(ctx_b, acc.at[dst_i], add=True)
        return carry

      lax.fori_loop(0, n_j, cnt_body, 0)
      plsc.subcore_barrier()
      pltpu.sync_copy(acc.at[pl.ds(sid * _RPT, _RPT)],
                      cnt_hbm.at[cid].at[pl.ds(wbase + sid * _RPT, _RPT)])
      plsc.subcore_barrier()
      return carry

    lax.fori_loop(0, _NWIN, cnt_win_body, 0)


  return edge_kernel


_edge_kernel_128 = _make_edge_kernel(128, 16, 384)
_edge_kernel_64 = _make_edge_kernel(64, 8, 256)


@jax.jit
def kernel(x, edge_index, batch,
           Wq0, bq0, Wk0, bk0, Wv0, bv0,
           Wq1, bq1, Wk1, bk1, Wv1, bv1,
           Wq2, bq2, Wk2, bk2, Wv2, bv2):
  del batch
  ei = edge_index.astype(jnp.int32)
  src_full = ei[0]
  dst_full = ei[1]
  keeps = [jnp.asarray(k) for k in _KEEP]

  W0, b0 = _prep_weights(Wq0, bq0, Wk0, bk0, Wv0, bv0, 384)
  W1, b1 = _prep_weights(Wq1, bq1, Wk1, bk1, Wv1, bv1, 256)
  W2, b2 = _prep_weights(Wq2, bq2, Wk2, bk2, Wv2, bv2, 384)

  xpad = jnp.pad(x, ((0, _N_PAD - _N_NODES), (0, 0)))

  u0, v0 = _proj(xpad, W0, b0)
  part0, cnt0, _ = _edge_kernel_128(u0, v0, src_full, dst_full, keeps[0])
  x0, u1, v1 = _fin_proj(part0, cnt0, W1, b1, 128)
  part1, cnt1, _ = _edge_kernel_64(u1, v1, src_full, dst_full, keeps[1])
  x1, u2, v2 = _fin_proj(part1, cnt1, W2, b2, 64)
  del x1
  part2, cnt2, _ = _edge_kernel_128(u2, v2, src_full, dst_full, keeps[2])
  out = _fin_res(part2, cnt2, x0)
  return out[:_N_NODES]


# speedup vs baseline: 1.1203x; 1.1203x over previous
import functools
import math

import numpy as np
import jax
import jax.numpy as jnp
from jax import lax
from jax.experimental import pallas as pl
from jax.experimental.pallas import tpu as pltpu
from jax.experimental.pallas import tpu_sc as plsc

_N_NODES = 10000
_N_EDGES = 320000
_E_KEEP = _N_EDGES // 2
_N_PAD = 10240
_CHUNK = 80
_N_CHUNKS = _E_KEEP // _CHUNK
_NW = 32
_BR = 1024
_WIN = 1280
_NWIN = _N_PAD // _WIN
_RPT = _WIN // 16

_KEEP = [
    np.asarray(np.random.default_rng(s).permutation(_N_EDGES)[:_E_KEEP],
               dtype=np.int32)
    for s in range(3)
]


def _prep_weights(Wq, bq, Wk, bk, Wv, bv, c3_pad):
  c_in = Wq.shape[1] // 2

  def split(W):
    Wi = W[:, :c_in]
    Wj = W[:, c_in:]
    return (Wi - Wj).T, Wj.T

  Aq, Bq = split(Wq)
  Ak, Bk = split(Wk)
  Av, Bv = split(Wv)
  pad = c3_pad - 3 * Wq.shape[0]
  zpad = jnp.zeros((c_in, pad), jnp.float32)
  Wcat = jnp.concatenate([Aq, Ak, Av, zpad, Bq, Bk, Bv, zpad], axis=1)
  bcat = jnp.concatenate(
      [bq, bk, bv, jnp.zeros((pad,), jnp.float32)]).reshape(1, -1)
  return Wcat, bcat


def _proj_body(x_ref, w_ref, b_ref, u_ref, v_ref):
  y = jnp.dot(x_ref[...], w_ref[...], preferred_element_type=jnp.float32)
  half = u_ref.shape[1]
  u_ref[...] = y[:, :half] + b_ref[...]
  v_ref[...] = y[:, half:]


def _proj(xpad, Wcat, bcat):
  c_in = xpad.shape[1]
  c3 = Wcat.shape[1] // 2
  grid = _N_PAD // _BR
  return pl.pallas_call(
      _proj_body,
      grid=(grid,),
      in_specs=[
          pl.BlockSpec((_BR, c_in), lambda i: (i, 0)),
          pl.BlockSpec((c_in, 2 * c3), lambda i: (0, 0)),
          pl.BlockSpec((1, c3), lambda i: (0, 0)),
      ],
      out_specs=[
          pl.BlockSpec((_BR, c3), lambda i: (i, 0)),
          pl.BlockSpec((_BR, c3), lambda i: (i, 0)),
      ],
      out_shape=[
          jax.ShapeDtypeStruct((_N_PAD, c3), jnp.float32),
          jax.ShapeDtypeStruct((_N_PAD, c3), jnp.float32),
      ],
  )(xpad, Wcat, bcat)


def _fin_proj_body(p_ref, c_ref, w_ref, b_ref, x_ref, u_ref, v_ref, *, cp):
  s = p_ref[0] + p_ref[1]
  cnt = c_ref[:, 0:1]
  h = jnp.where(cnt > 0, s[:, :cp] / jnp.maximum(cnt, 1.0), 0.0)
  h = jnp.maximum(h, 0.0)
  x_ref[...] = h
  y = jnp.dot(h, w_ref[...], preferred_element_type=jnp.float32)
  half = u_ref.shape[1]
  u_ref[...] = y[:, :half] + b_ref[...]
  v_ref[...] = y[:, half:]


def _fin_proj(part, cnt, Wcat, bcat, cp):
  c3 = Wcat.shape[1] // 2
  grid = _N_PAD // _BR
  return pl.pallas_call(
      functools.partial(_fin_proj_body, cp=cp),
      grid=(grid,),
      in_specs=[
          pl.BlockSpec((2, _BR, 128), lambda i: (0, i, 0)),
          pl.BlockSpec((_BR, 128), lambda i: (i, 0)),
          pl.BlockSpec((cp, 2 * c3), lambda i: (0, 0)),
          pl.BlockSpec((1, c3), lambda i: (0, 0)),
      ],
      out_specs=[
          pl.BlockSpec((_BR, cp), lambda i: (i, 0)),
          pl.BlockSpec((_BR, c3), lambda i: (i, 0)),
          pl.BlockSpec((_BR, c3), lambda i: (i, 0)),
      ],
      out_shape=[
          jax.ShapeDtypeStruct((_N_PAD, cp), jnp.float32),
          jax.ShapeDtypeStruct((_N_PAD, c3), jnp.float32),
          jax.ShapeDtypeStruct((_N_PAD, c3), jnp.float32),
      ],
  )(part, cnt, Wcat, bcat)


def _fin_res_body(p_ref, c_ref, x0_ref, o_ref):
  s = p_ref[0] + p_ref[1]
  cnt = c_ref[:, 0:1]
  h = jnp.where(cnt > 0, s / jnp.maximum(cnt, 1.0), 0.0)
  o_ref[...] = jnp.maximum(h + x0_ref[...], 0.0)


def _fin_res(part, cnt, x0):
  grid = _N_PAD // _BR
  return pl.pallas_call(
      _fin_res_body,
      grid=(grid,),
      in_specs=[
          pl.BlockSpec((2, _BR, 128), lambda i: (0, i, 0)),
          pl.BlockSpec((_BR, 128), lambda i: (i, 0)),
          pl.BlockSpec((_BR, 128), lambda i: (i, 0)),
      ],
      out_specs=pl.BlockSpec((_BR, 128), lambda i: (i, 0)),
      out_shape=jax.ShapeDtypeStruct((_N_PAD, 128), jnp.float32),
  )(part, cnt, x0)


def _make_edge_kernel(c_out, hs, c3_pad):
  n_vreg = c_out // 16
  inv = 1.0 / math.sqrt(hs)
  mesh = plsc.VectorSubcoreMesh(core_axis_name="c", subcore_axis_name="s")

  @functools.partial(
      pl.kernel,
      mesh=mesh,
      out_type=[
          jax.ShapeDtypeStruct((2, _N_PAD, 128), jnp.float32),
          jax.ShapeDtypeStruct((2, _N_PAD, 128), jnp.float32),
          jax.ShapeDtypeStruct((_E_KEEP, 128), jnp.float32),
      ],
      scratch_types=[
          pltpu.VMEM((_CHUNK,), jnp.int32),
          pltpu.VMEM((_CHUNK,), jnp.int32),
          pltpu.VMEM((_CHUNK,), jnp.int32),
          pltpu.VMEM((_CHUNK,), jnp.int32),
          pltpu.VMEM((_CHUNK, c3_pad), jnp.float32),
          pltpu.VMEM((_CHUNK, c3_pad), jnp.float32),
          pltpu.VMEM((_CHUNK, 128), jnp.float32),
          pltpu.VMEM((_CHUNK + 16,), jnp.float32),
          pltpu.VMEM((_N_PAD + 16,), jnp.float32),
          pltpu.VMEM_SHARED((_WIN + 8, 128), jnp.float32),
          pltpu.SemaphoreType.DMA,
          pltpu.SemaphoreType.DMA,
      ],
  )
  def edge_kernel(u_hbm, v_hbm, src_hbm, dst_hbm, keep_hbm,
                  part_hbm, cnt_hbm, ctx_hbm,
                  kpos_v, src_v, dst_v, dst_i, u_b, v_b, ctx_b, w_b, cnt_v, acc,
                  sem1, sem2):
    cid = lax.axis_index("c")
    sid = lax.axis_index("s")
    wid = cid * 16 + sid
    zero16 = jnp.zeros((16,), jnp.float32)
    onesv = jnp.where(lax.iota(jnp.int32, 16) == 0,
                      jnp.float32(1.0), jnp.float32(0.0))

    def zrow(r, carry):
      for c in range(8):
        ctx_b[r, pl.ds(c * 16, 16)] = zero16
      return carry

    def zero_acc():
      pltpu.sync_copy(ctx_b, acc.at[pl.ds(sid * _RPT, _CHUNK)])
      @pl.when(sid == 15)
      def _():
        pltpu.sync_copy(ctx_b.at[pl.ds(0, 8)], acc.at[pl.ds(_WIN, 8)])

    def zcnt(r, carry):
      cnt_v[pl.ds(r * 16, 16)] = zero16
      return carry

    lax.fori_loop(0, (_N_PAD + 16) // 16, zcnt, 0)

    n_j = jnp.where(wid < _N_CHUNKS % _NW,
                    _N_CHUNKS // _NW + 1, _N_CHUNKS // _NW)

    lane = lax.iota(jnp.int32, 16)
    perm_idx = []
    st = hs // 2
    while st >= 1:
      perm_idx.append(jnp.bitwise_xor(lane, st))
      st //= 2

    def local_idx(wbase):
      for g in range(_CHUNK // 16):
        dv = dst_v[pl.ds(g * 16, 16)]
        loc = dv - wbase
        ok = jnp.logical_and(loc >= 0, loc < _WIN)
        dst_i[pl.ds(g * 16, 16)] = jnp.where(ok, loc, _WIN)

    def edge_body(e, ecarry):
      w = w_b[pl.ds(e, 16)][0]
      d = dst_v[pl.ds(e, 16)][0]
      cnt_v[pl.ds(d, 16)] = cnt_v[pl.ds(d, 16)] + onesv
      for h in range(n_vreg):
        q = u_b[e, pl.ds(h * 16, 16)] + v_b[e, pl.ds(h * 16, 16)]
        k = (u_b[e, pl.ds(c_out + h * 16, 16)]
             + v_b[e, pl.ds(c_out + h * 16, 16)])
        v = (u_b[e, pl.ds(2 * c_out + h * 16, 16)]
             + v_b[e, pl.ds(2 * c_out + h * 16, 16)])
        s = q * k * inv
        m = s
        for ix in perm_idx:
          m = jnp.maximum(m, m.at[ix].get(mode="promise_in_bounds"))
        p = jnp.exp(s - m)
        z = p
        for ix in perm_idx:
          z = z + z.at[ix].get(mode="promise_in_bounds")
        ctx_b[e, pl.ds(h * 16, 16)] = p * v * (w / z)
      return ecarry

    def win_body(w, carry):
      wbase = pl.multiple_of(w * _WIN, _WIN)
      lax.fori_loop(0, _CHUNK, zrow, 0)
      zero_acc()
      plsc.subcore_barrier()

      def chunk_body(j, carry):
        base = (j * _NW + wid) * _CHUNK
        pltpu.sync_copy(keep_hbm.at[pl.ds(base, _CHUNK)], kpos_v)
        pltpu.async_copy(dst_hbm.at[kpos_v], dst_v, sem2).wait()

        @pl.when(w == 0)
        def _():
          pltpu.async_copy(src_hbm.at[kpos_v], src_v, sem1).wait()
          h1 = pltpu.async_copy(u_hbm.at[dst_v], u_b, sem1)
          h2 = pltpu.async_copy(v_hbm.at[src_v], v_b, sem2)
          h1.wait()
          h2.wait()
          for g in range(_CHUNK // 16):
            sv = src_v[pl.ds(g * 16, 16)]
            dv = dst_v[pl.ds(g * 16, 16)]
            dvec = jnp.abs(dv - sv)
            w_b[pl.ds(g * 16, 16)] = jnp.where(
                dvec > 8, jnp.float32(1.0),
                jnp.where(dvec < 8, jnp.float32(-1.0), jnp.float32(0.0)))
          lax.fori_loop(0, _CHUNK, edge_body, 0)
          pltpu.sync_copy(ctx_b, ctx_hbm.at[pl.ds(base, _CHUNK)])

        @pl.when(w != 0)
        def _():
          pltpu.sync_copy(ctx_hbm.at[pl.ds(base, _CHUNK)], ctx_b)

        local_idx(wbase)
        pltpu.sync_copy(ctx_b, acc.at[dst_i], add=True)
        return carry

      lax.fori_loop(0, n_j, chunk_body, 0)
      plsc.subcore_barrier()
      pltpu.sync_copy(acc.at[pl.ds(sid * _RPT, _RPT)],
                      part_hbm.at[cid].at[pl.ds(wbase + sid * _RPT, _RPT)])
      plsc.subcore_barrier()
      return carry

    lax.fori_loop(0, _NWIN, win_body, 0)

    def cdump(r, carry):
      pltpu.sync_copy(cnt_v.at[pl.ds(r * 128, 128)],
                      cnt_hbm.at[cid].at[sid * 80 + r])
      return carry

    lax.fori_loop(0, _N_PAD // 128, cdump, 0)


  return edge_kernel


_edge_kernel_128 = _make_edge_kernel(128, 16, 384)
_edge_kernel_64 = _make_edge_kernel(64, 8, 256)


@jax.jit
def kernel(x, edge_index, batch,
           Wq0, bq0, Wk0, bk0, Wv0, bv0,
           Wq1, bq1, Wk1, bk1, Wv1, bv1,
           Wq2, bq2, Wk2, bk2, Wv2, bv2):
  del batch
  ei = edge_index.astype(jnp.int32)
  src_full = ei[0]
  dst_full = ei[1]
  keeps = [jnp.asarray(k) for k in _KEEP]

  W0, b0 = _prep_weights(Wq0, bq0, Wk0, bk0, Wv0, bv0, 384)
  W1, b1 = _prep_weights(Wq1, bq1, Wk1, bk1, Wv1, bv1, 256)
  W2, b2 = _prep_weights(Wq2, bq2, Wk2, bk2, Wv2, bv2, 384)

  xpad = jnp.pad(x, ((0, _N_PAD - _N_NODES), (0, 0)))

  def fold_counts(cnt_raw):
    c = cnt_raw[:, :1280, :].reshape(2, 16, 80, 128).sum((0, 1))
    return jnp.broadcast_to(c.reshape(_N_PAD)[:, None], (_N_PAD, 128))

  u0, v0 = _proj(xpad, W0, b0)
  part0, cnt0, _ = _edge_kernel_128(u0, v0, src_full, dst_full, keeps[0])
  x0, u1, v1 = _fin_proj(part0, fold_counts(cnt0), W1, b1, 128)
  part1, cnt1, _ = _edge_kernel_64(u1, v1, src_full, dst_full, keeps[1])
  x1, u2, v2 = _fin_proj(part1, fold_counts(cnt1), W2, b2, 64)
  del x1
  part2, cnt2, _ = _edge_kernel_128(u2, v2, src_full, dst_full, keeps[2])
  out = _fin_res(part2, fold_counts(cnt2), x0)
  return out[:_N_NODES]


# speedup vs baseline: 1.3426x; 1.1984x over previous
import functools
import math

import numpy as np
import jax
import jax.numpy as jnp
from jax import lax
from jax.experimental import pallas as pl
from jax.experimental.pallas import tpu as pltpu
from jax.experimental.pallas import tpu_sc as plsc

_N_NODES = 10000
_N_EDGES = 320000
_E_KEEP = _N_EDGES // 2
_N_PAD = 10240
_CHUNK = 80
_N_CHUNKS = _E_KEEP // _CHUNK
_NW = 32
_BR = 1024
_WIN = 2048
_NWIN = _N_PAD // _WIN
_RPT = _WIN // 16

_KEEP = [
    np.asarray(np.random.default_rng(s).permutation(_N_EDGES)[:_E_KEEP],
               dtype=np.int32)
    for s in range(3)
]


def _prep_weights(Wq, bq, Wk, bk, Wv, bv, c3_pad):
  c_in = Wq.shape[1] // 2

  def split(W):
    Wi = W[:, :c_in]
    Wj = W[:, c_in:]
    return (Wi - Wj).T, Wj.T

  Aq, Bq = split(Wq)
  Ak, Bk = split(Wk)
  Av, Bv = split(Wv)
  pad = c3_pad - 3 * Wq.shape[0]
  zpad = jnp.zeros((c_in, pad), jnp.float32)
  Wcat = jnp.concatenate([Aq, Ak, Av, zpad, Bq, Bk, Bv, zpad], axis=1)
  bcat = jnp.concatenate(
      [bq, bk, bv, jnp.zeros((pad,), jnp.float32)]).reshape(1, -1)
  return Wcat, bcat


def _proj_body(x_ref, w_ref, b_ref, u_ref, v_ref):
  y = jnp.dot(x_ref[...], w_ref[...], preferred_element_type=jnp.float32)
  half = u_ref.shape[1]
  u_ref[...] = y[:, :half] + b_ref[...]
  v_ref[...] = y[:, half:]


def _proj(xpad, Wcat, bcat):
  c_in = xpad.shape[1]
  c3 = Wcat.shape[1] // 2
  grid = _N_PAD // _BR
  return pl.pallas_call(
      _proj_body,
      grid=(grid,),
      in_specs=[
          pl.BlockSpec((_BR, c_in), lambda i: (i, 0)),
          pl.BlockSpec((c_in, 2 * c3), lambda i: (0, 0)),
          pl.BlockSpec((1, c3), lambda i: (0, 0)),
      ],
      out_specs=[
          pl.BlockSpec((_BR, c3), lambda i: (i, 0)),
          pl.BlockSpec((_BR, c3), lambda i: (i, 0)),
      ],
      out_shape=[
          jax.ShapeDtypeStruct((_N_PAD, c3), jnp.float32),
          jax.ShapeDtypeStruct((_N_PAD, c3), jnp.float32),
      ],
  )(xpad, Wcat, bcat)


def _fin_proj_body(p_ref, c_ref, w_ref, b_ref, x_ref, u_ref, v_ref, *, cp):
  s = p_ref[0] + p_ref[1]
  cnt = c_ref[:, 0:1]
  h = jnp.where(cnt > 0, s[:, :cp] / jnp.maximum(cnt, 1.0), 0.0)
  h = jnp.maximum(h, 0.0)
  x_ref[...] = h
  y = jnp.dot(h, w_ref[...], preferred_element_type=jnp.float32)
  half = u_ref.shape[1]
  u_ref[...] = y[:, :half] + b_ref[...]
  v_ref[...] = y[:, half:]


def _fin_proj(part, cnt, Wcat, bcat, cp):
  c3 = Wcat.shape[1] // 2
  grid = _N_PAD // _BR
  return pl.pallas_call(
      functools.partial(_fin_proj_body, cp=cp),
      grid=(grid,),
      in_specs=[
          pl.BlockSpec((2, _BR, 128), lambda i: (0, i, 0)),
          pl.BlockSpec((_BR, 128), lambda i: (i, 0)),
          pl.BlockSpec((cp, 2 * c3), lambda i: (0, 0)),
          pl.BlockSpec((1, c3), lambda i: (0, 0)),
      ],
      out_specs=[
          pl.BlockSpec((_BR, cp), lambda i: (i, 0)),
          pl.BlockSpec((_BR, c3), lambda i: (i, 0)),
          pl.BlockSpec((_BR, c3), lambda i: (i, 0)),
      ],
      out_shape=[
          jax.ShapeDtypeStruct((_N_PAD, cp), jnp.float32),
          jax.ShapeDtypeStruct((_N_PAD, c3), jnp.float32),
          jax.ShapeDtypeStruct((_N_PAD, c3), jnp.float32),
      ],
  )(part, cnt, Wcat, bcat)


def _fin_res_body(p_ref, c_ref, x0_ref, o_ref):
  s = p_ref[0] + p_ref[1]
  cnt = c_ref[:, 0:1]
  h = jnp.where(cnt > 0, s / jnp.maximum(cnt, 1.0), 0.0)
  o_ref[...] = jnp.maximum(h + x0_ref[...], 0.0)


def _fin_res(part, cnt, x0):
  grid = _N_PAD // _BR
  return pl.pallas_call(
      _fin_res_body,
      grid=(grid,),
      in_specs=[
          pl.BlockSpec((2, _BR, 128), lambda i: (0, i, 0)),
          pl.BlockSpec((_BR, 128), lambda i: (i, 0)),
          pl.BlockSpec((_BR, 128), lambda i: (i, 0)),
      ],
      out_specs=pl.BlockSpec((_BR, 128), lambda i: (i, 0)),
      out_shape=jax.ShapeDtypeStruct((_N_PAD, 128), jnp.float32),
  )(part, cnt, x0)


def _make_edge_kernel(c_out, hs, c3_pad):
  n_vreg = c_out // 16
  inv = 1.0 / math.sqrt(hs)
  mesh = plsc.VectorSubcoreMesh(core_axis_name="c", subcore_axis_name="s")

  @functools.partial(
      pl.kernel,
      mesh=mesh,
      out_type=[
          jax.ShapeDtypeStruct((2, _N_PAD, 128), jnp.float32),
          jax.ShapeDtypeStruct((2, _N_PAD, 128), jnp.float32),
          jax.ShapeDtypeStruct((_E_KEEP, 128), jnp.float32),
      ],
      scratch_types=[
          pltpu.VMEM((_CHUNK,), jnp.int32),
          pltpu.VMEM((_CHUNK,), jnp.int32),
          pltpu.VMEM((_CHUNK,), jnp.int32),
          pltpu.VMEM((_CHUNK,), jnp.int32),
          pltpu.VMEM((_CHUNK, c3_pad), jnp.float32),
          pltpu.VMEM((_CHUNK, c3_pad), jnp.float32),
          pltpu.VMEM((_CHUNK, 128), jnp.float32),
          pltpu.VMEM((_CHUNK + 16,), jnp.float32),
          pltpu.VMEM((_N_PAD + 16,), jnp.float32),
          pltpu.VMEM_SHARED((_WIN + 8, 128), jnp.float32),
          pltpu.SemaphoreType.DMA,
          pltpu.SemaphoreType.DMA,
      ],
  )
  def edge_kernel(u_hbm, v_hbm, src_hbm, dst_hbm, keep_hbm,
                  part_hbm, cnt_hbm, ctx_hbm,
                  kpos_v, src_v, dst_v, dst_i, u_b, v_b, ctx_b, w_b, cnt_v, acc,
                  sem1, sem2):
    cid = lax.axis_index("c")
    sid = lax.axis_index("s")
    wid = cid * 16 + sid
    zero16 = jnp.zeros((16,), jnp.float32)
    onesv = jnp.where(lax.iota(jnp.int32, 16) == 0,
                      jnp.float32(1.0), jnp.float32(0.0))

    def zrow(r, carry):
      for c in range(8):
        ctx_b[r, pl.ds(c * 16, 16)] = zero16
      return carry

    def zero_acc():
      pltpu.sync_copy(ctx_b, acc.at[pl.ds(sid * _RPT, _CHUNK)])
      pltpu.sync_copy(ctx_b.at[pl.ds(0, _RPT - _CHUNK)],
                      acc.at[pl.ds(sid * _RPT + _CHUNK, _RPT - _CHUNK)])
      @pl.when(sid == 15)
      def _():
        pltpu.sync_copy(ctx_b.at[pl.ds(0, 8)], acc.at[pl.ds(_WIN, 8)])

    def zcnt(r, carry):
      cnt_v[pl.ds(r * 16, 16)] = zero16
      return carry

    lax.fori_loop(0, (_N_PAD + 16) // 16, zcnt, 0)

    n_j = jnp.where(wid < _N_CHUNKS % _NW,
                    _N_CHUNKS // _NW + 1, _N_CHUNKS // _NW)

    lane = lax.iota(jnp.int32, 16)
    perm_idx = []
    st = hs // 2
    while st >= 1:
      perm_idx.append(jnp.bitwise_xor(lane, st))
      st //= 2

    def local_idx(wbase):
      for g in range(_CHUNK // 16):
        dv = dst_v[pl.ds(g * 16, 16)]
        loc = dv - wbase
        ok = jnp.logical_and(loc >= 0, loc < _WIN)
        dst_i[pl.ds(g * 16, 16)] = jnp.where(ok, loc, _WIN)

    def edge_body(e, ecarry):
      w = w_b[pl.ds(e, 16)][0]
      d = dst_v[pl.ds(e, 16)][0]
      cnt_v[pl.ds(d, 16)] = cnt_v[pl.ds(d, 16)] + onesv
      for h in range(n_vreg):
        q = u_b[e, pl.ds(h * 16, 16)] + v_b[e, pl.ds(h * 16, 16)]
        k = (u_b[e, pl.ds(c_out + h * 16, 16)]
             + v_b[e, pl.ds(c_out + h * 16, 16)])
        v = (u_b[e, pl.ds(2 * c_out + h * 16, 16)]
             + v_b[e, pl.ds(2 * c_out + h * 16, 16)])
        s = q * k * inv
        m = s
        for ix in perm_idx:
          m = jnp.maximum(m, m.at[ix].get(mode="promise_in_bounds"))
        p = jnp.exp(s - m)
        z = p
        for ix in perm_idx:
          z = z + z.at[ix].get(mode="promise_in_bounds")
        ctx_b[e, pl.ds(h * 16, 16)] = p * v * (w / z)
      return ecarry

    def win_body(w, carry):
      wbase = pl.multiple_of(w * _WIN, _WIN)
      lax.fori_loop(0, _CHUNK, zrow, 0)
      zero_acc()
      plsc.subcore_barrier()

      def chunk_body(j, carry):
        base = (j * _NW + wid) * _CHUNK
        pltpu.sync_copy(keep_hbm.at[pl.ds(base, _CHUNK)], kpos_v)
        pltpu.async_copy(dst_hbm.at[kpos_v], dst_v, sem2).wait()

        @pl.when(w == 0)
        def _():
          pltpu.async_copy(src_hbm.at[kpos_v], src_v, sem1).wait()
          h1 = pltpu.async_copy(u_hbm.at[dst_v], u_b, sem1)
          h2 = pltpu.async_copy(v_hbm.at[src_v], v_b, sem2)
          h1.wait()
          h2.wait()
          for g in range(_CHUNK // 16):
            sv = src_v[pl.ds(g * 16, 16)]
            dv = dst_v[pl.ds(g * 16, 16)]
            dvec = jnp.abs(dv - sv)
            w_b[pl.ds(g * 16, 16)] = jnp.where(
                dvec > 8, jnp.float32(1.0),
                jnp.where(dvec < 8, jnp.float32(-1.0), jnp.float32(0.0)))
          lax.fori_loop(0, _CHUNK, edge_body, 0)
          pltpu.sync_copy(ctx_b, ctx_hbm.at[pl.ds(base, _CHUNK)])

        @pl.when(w != 0)
        def _():
          pltpu.sync_copy(ctx_hbm.at[pl.ds(base, _CHUNK)], ctx_b)

        local_idx(wbase)
        pltpu.sync_copy(ctx_b, acc.at[dst_i], add=True)
        return carry

      lax.fori_loop(0, n_j, chunk_body, 0)
      plsc.subcore_barrier()
      pltpu.sync_copy(acc.at[pl.ds(sid * _RPT, _RPT)],
                      part_hbm.at[cid].at[pl.ds(wbase + sid * _RPT, _RPT)])
      plsc.subcore_barrier()
      return carry

    lax.fori_loop(0, _NWIN, win_body, 0)

    def cdump(r, carry):
      pltpu.sync_copy(cnt_v.at[pl.ds(r * 128, 128)],
                      cnt_hbm.at[cid].at[sid * 80 + r])
      return carry

    lax.fori_loop(0, _N_PAD // 128, cdump, 0)


  return edge_kernel


_edge_kernel_128 = _make_edge_kernel(128, 16, 384)
_edge_kernel_64 = _make_edge_kernel(64, 8, 256)


@jax.jit
def kernel(x, edge_index, batch,
           Wq0, bq0, Wk0, bk0, Wv0, bv0,
           Wq1, bq1, Wk1, bk1, Wv1, bv1,
           Wq2, bq2, Wk2, bk2, Wv2, bv2):
  del batch
  ei = edge_index.astype(jnp.int32)
  src_full = ei[0]
  dst_full = ei[1]
  keeps = [jnp.asarray(k) for k in _KEEP]

  W0, b0 = _prep_weights(Wq0, bq0, Wk0, bk0, Wv0, bv0, 384)
  W1, b1 = _prep_weights(Wq1, bq1, Wk1, bk1, Wv1, bv1, 256)
  W2, b2 = _prep_weights(Wq2, bq2, Wk2, bk2, Wv2, bv2, 384)

  xpad = jnp.pad(x, ((0, _N_PAD - _N_NODES), (0, 0)))

  def fold_counts(cnt_raw):
    c = cnt_raw[:, :1280, :].reshape(2, 16, 80, 128).sum((0, 1))
    return jnp.broadcast_to(c.reshape(_N_PAD)[:, None], (_N_PAD, 128))

  u0, v0 = _proj(xpad, W0, b0)
  part0, cnt0, _ = _edge_kernel_128(u0, v0, src_full, dst_full, keeps[0])
  x0, u1, v1 = _fin_proj(part0, fold_counts(cnt0), W1, b1, 128)
  part1, cnt1, _ = _edge_kernel_64(u1, v1, src_full, dst_full, keeps[1])
  x1, u2, v2 = _fin_proj(part1, fold_counts(cnt1), W2, b2, 64)
  del x1
  part2, cnt2, _ = _edge_kernel_128(u2, v2, src_full, dst_full, keeps[2])
  out = _fin_res(part2, fold_counts(cnt2), x0)
  return out[:_N_NODES]
